# baseline (speedup 1.0000x reference)
import functools

import jax
import jax.numpy as jnp
from jax import lax
from jax.experimental import pallas as pl
from jax.experimental.pallas import tpu as pltpu
from jax.experimental.pallas import tpu_sc as plsc

SEQ = 4096
B = 4
D = 1024
LANES = 16
NC = 2
NS = 16
NW = NC * NS
ROWS_PER_W = SEQ // NW
C = 8
NCH = ROWS_PER_W // C
VECS = D // LANES


def _sc_body(x_hbm, pos_hbm, out_hbm, x_v, pos_v, sin0, sin1, sout0, sout1):
    wid = lax.axis_index("s") * NC + lax.axis_index("c")
    base = wid * ROWS_PER_W
    sems_in = (sin0, sin1)
    sems_out = (sout0, sout1)

    def start_in(ci, buf):
        r0 = base + ci * C
        hx = pltpu.async_copy(x_hbm.at[pl.ds(r0, C)], x_v.at[buf], sems_in[buf])
        hp = pltpu.async_copy(pos_hbm.at[pl.ds(r0, C)], pos_v.at[buf], sems_in[buf])
        return (hx, hp)

    def start_out(ci, buf):
        r0 = base + ci * C
        return pltpu.async_copy(x_v.at[buf], out_hbm.at[pl.ds(r0, C)], sems_out[buf])

    def compute(buf):
        def body(j, carry):
            r = j // VECS
            off = (j - r * VECS) * LANES
            p = pos_v[buf, r, pl.ds(off, LANES)]
            for bb in range(B):
                x_v[buf, r, bb, pl.ds(off, LANES)] += p
            return carry
        lax.fori_loop(0, C * VECS, body, 0)

    pending_in = start_in(0, 0)
    pending_out = [None, None]
    for ci in range(NCH):
        buf = ci & 1
        nbuf = buf ^ 1
        next_in = None
        if ci + 1 < NCH:
            if pending_out[nbuf] is not None:
                pending_out[nbuf].wait()
                pending_out[nbuf] = None
            next_in = start_in(ci + 1, nbuf)
        for h in pending_in:
            h.wait()
        compute(buf)
        pending_out[buf] = start_out(ci, buf)
        if next_in is not None:
            pending_in = next_in
    for h in pending_out:
        if h is not None:
            h.wait()


@functools.partial(
    pl.kernel,
    out_type=jax.ShapeDtypeStruct((SEQ, B, D), jnp.float32),
    mesh=plsc.VectorSubcoreMesh(core_axis_name="c", subcore_axis_name="s"),
    scratch_types=[
        pltpu.VMEM((2, C, B, D), jnp.float32),
        pltpu.VMEM((2, C, D), jnp.float32),
        pltpu.SemaphoreType.DMA,
        pltpu.SemaphoreType.DMA,
        pltpu.SemaphoreType.DMA,
        pltpu.SemaphoreType.DMA,
    ],
)
def _sc_add(x_hbm, pos_hbm, out_hbm, x_v, pos_v, sin0, sin1, sout0, sout1):
    _sc_body(x_hbm, pos_hbm, out_hbm, x_v, pos_v, sin0, sin1, sout0, sout1)


def kernel(x, pos_table):
    return _sc_add(x, pos_table)


# speedup vs baseline: 1.0009x; 1.0009x over previous
import functools

import jax
import jax.numpy as jnp
from jax import lax
from jax.experimental import pallas as pl
from jax.experimental.pallas import tpu as pltpu
from jax.experimental.pallas import tpu_sc as plsc

SEQ = 4096
B = 4
D = 1024
LANES = 16
NC = 2
NS = 16
NW = NC * NS
ROWS_PER_W = SEQ // NW
C = 8
NCH = ROWS_PER_W // C
VECS = D // LANES


def _sc_body(x_hbm, pos_hbm, out_hbm, x_v, pos_v, sin0, sin1, sout0, sout1):
    wid = lax.axis_index("s") * NC + lax.axis_index("c")
    base = wid * ROWS_PER_W
    sems_in = (sin0, sin1)
    sems_out = (sout0, sout1)

    def start_in(ci, buf):
        r0 = base + ci * C
        hx = pltpu.async_copy(x_hbm.at[pl.ds(r0, C)], x_v.at[buf], sems_in[buf])
        hp = pltpu.async_copy(pos_hbm.at[pl.ds(r0, C)], pos_v.at[buf], sems_in[buf])
        return (hx, hp)

    def start_out(ci, buf):
        r0 = base + ci * C
        return pltpu.async_copy(x_v.at[buf], out_hbm.at[pl.ds(r0, C)], sems_out[buf])

    def compute(buf):
        def body(j, carry):
            r = j // VECS
            off = (j - r * VECS) * LANES
            p = pos_v[buf, r, pl.ds(off, LANES)]
            for bb in range(B):
                x_v[buf, r, bb, pl.ds(off, LANES)] += p
            return carry
        lax.fori_loop(0, C * VECS, body, 0, unroll=8)

    pending_in = start_in(0, 0)
    pending_out = [None, None]
    for ci in range(NCH):
        buf = ci & 1
        nbuf = buf ^ 1
        next_in = None
        if ci + 1 < NCH:
            if pending_out[nbuf] is not None:
                pending_out[nbuf].wait()
                pending_out[nbuf] = None
            next_in = start_in(ci + 1, nbuf)
        for h in pending_in:
            h.wait()
        compute(buf)
        pending_out[buf] = start_out(ci, buf)
        if next_in is not None:
            pending_in = next_in
    for h in pending_out:
        if h is not None:
            h.wait()


@functools.partial(
    pl.kernel,
    out_type=jax.ShapeDtypeStruct((SEQ, B, D), jnp.float32),
    mesh=plsc.VectorSubcoreMesh(core_axis_name="c", subcore_axis_name="s"),
    scratch_types=[
        pltpu.VMEM((2, C, B, D), jnp.float32),
        pltpu.VMEM((2, C, D), jnp.float32),
        pltpu.SemaphoreType.DMA,
        pltpu.SemaphoreType.DMA,
        pltpu.SemaphoreType.DMA,
        pltpu.SemaphoreType.DMA,
    ],
)
def _sc_add(x_hbm, pos_hbm, out_hbm, x_v, pos_v, sin0, sin1, sout0, sout1):
    _sc_body(x_hbm, pos_hbm, out_hbm, x_v, pos_v, sin0, sin1, sout0, sout1)


def kernel(x, pos_table):
    return _sc_add(x, pos_table)
